# baseline (speedup 1.0000x reference)
import functools

import jax
import jax.numpy as jnp
from jax import lax
from jax.experimental import pallas as pl
from jax.experimental.pallas import tpu as pltpu
from jax.experimental.pallas import tpu_sc as plsc

_D = 1024
_NW = 32
_NC = 2
_CH = 16
_NBUF = 4


def _emb_body(idx_hbm, table_hbm, out_hbm, idx_v, rows_v,
              g0, g1, g2, g3, s0, s1, s2, s3):
    gsems = (g0, g1, g2, g3)
    ssems = (s0, s1, s2, s3)
    bpw = idx_hbm.shape[0] // _NW
    nchunk = bpw // _CH
    wid = lax.axis_index("s") * _NC + lax.axis_index("c")
    base = wid * bpw

    pltpu.sync_copy(idx_hbm.at[pl.ds(base, bpw)], idx_v)

    def gather(c, bf):
        return pltpu.make_async_copy(
            table_hbm.at[idx_v.at[pl.ds(c * _CH, _CH)]], rows_v.at[bf],
            gsems[bf])

    def scat(c, bf):
        return pltpu.make_async_copy(
            rows_v.at[bf], out_hbm.at[pl.ds(base + c * _CH, _CH)], ssems[bf])

    def produce(c, bf):
        gather(c, bf).wait()
        scat(c, bf).start()

    def recycle(c, bf):
        scat(c, bf).wait()
        gather(c + _NBUF, bf).start()

    for c in range(_NBUF):
        gather(c, c).start()
    produce(0, 0)
    produce(1, 1)

    def group(g, carry):
        for bf in range(_NBUF):
            c = g * _NBUF + bf + 2
            recycle(c - 2, bf)
            produce(c, (bf + 2) % _NBUF)
        return carry

    lax.fori_loop(0, (nchunk - _NBUF) // _NBUF, group, 0)

    produce(nchunk - 2, (nchunk - 2) % _NBUF)
    produce(nchunk - 1, (nchunk - 1) % _NBUF)
    for c in range(nchunk - _NBUF, nchunk):
        scat(c, c % _NBUF).wait()


def kernel(position_ids, weight):
    batch, seq = position_ids.shape
    b = batch * seq
    idx = position_ids.reshape(b).astype(jnp.int32)

    mesh = plsc.VectorSubcoreMesh(core_axis_name="c", subcore_axis_name="s")
    bpw = b // _NW

    run = functools.partial(
        pl.kernel,
        mesh=mesh,
        out_type=jax.ShapeDtypeStruct((b, _D), jnp.float32),
        scratch_types=[
            pltpu.VMEM((bpw,), jnp.int32),
            pltpu.VMEM((_NBUF, _CH, _D), jnp.float32),
        ] + [pltpu.SemaphoreType.DMA] * 8,
    )(_emb_body)

    out = run(idx, weight)
    return out.reshape(batch, seq, _D)


# speedup vs baseline: 1.0071x; 1.0071x over previous
import functools

import jax
import jax.numpy as jnp
from jax import lax
from jax.experimental import pallas as pl
from jax.experimental.pallas import tpu as pltpu
from jax.experimental.pallas import tpu_sc as plsc

_D = 1024
_NW = 32
_NC = 2
_CH = 16


_NBUF = 4


def _emb_body(idx_hbm, table_hbm, out_hbm, idx_v, rows_v,
              g0, g1, g2, g3, s0, s1, s2, s3):
    gsems = (g0, g1, g2, g3)
    ssems = (s0, s1, s2, s3)
    bpw = idx_hbm.shape[0] // _NW
    nchunk = bpw // _CH
    ngroup = nchunk // _NBUF
    wid = lax.axis_index("s") * _NC + lax.axis_index("c")
    base = wid * bpw

    pltpu.sync_copy(idx_hbm.at[pl.ds(base, bpw)], idx_v)

    def gather(c, bf):
        return pltpu.make_async_copy(
            table_hbm.at[idx_v.at[pl.ds(c * _CH, _CH)]], rows_v.at[bf],
            gsems[bf])

    def scatter(c, bf):
        return pltpu.make_async_copy(
            rows_v.at[bf], out_hbm.at[pl.ds(base + c * _CH, _CH)], ssems[bf])

    for bf in range(_NBUF):
        gather(bf, bf).start()

    def group(g, carry):
        for bf in range(_NBUF):
            c = g * _NBUF + bf
            gather(c, bf).wait()
            scatter(c, bf).start()
            scatter(c, bf).wait()
            gather(c + _NBUF, bf).start()
        return carry

    lax.fori_loop(0, ngroup - 1, group, 0)

    for bf in range(_NBUF):
        c = (ngroup - 1) * _NBUF + bf
        gather(c, bf).wait()
        scatter(c, bf).start()
        scatter(c, bf).wait()


def kernel(position_ids, weight):
    batch, seq = position_ids.shape
    b = batch * seq
    idx = position_ids.reshape(b).astype(jnp.int32)

    mesh = plsc.VectorSubcoreMesh(core_axis_name="c", subcore_axis_name="s")
    bpw = b // _NW

    run = functools.partial(
        pl.kernel,
        mesh=mesh,
        out_type=jax.ShapeDtypeStruct((b, _D), jnp.float32),
        scratch_types=[
            pltpu.VMEM((bpw,), jnp.int32),
            pltpu.VMEM((_NBUF, _CH, _D), jnp.float32),
        ] + [pltpu.SemaphoreType.DMA] * 8,
    )(_emb_body)

    out = run(idx, weight)
    return out.reshape(batch, seq, _D)
